# Initial kernel scaffold; baseline (speedup 1.0000x reference)
import functools

import jax
import jax.numpy as jnp
from jax import lax
from jax.experimental import pallas as pl
from jax.experimental.pallas import tpu as pltpu
from jax.experimental.pallas import tpu_sc as plsc

N_NODES = 10000
D = 128
N_EDGES = 320000
NC = 2
NS = 16
NW = NC * NS
L = 16
CH = 128
NCHUNK = (N_EDGES + NW * CH - 1) // (NW * CH)
EPT = NCHUNK * CH
E_PAD = NW * EPT
P = 10112
RPT = P // NS


def _sc_scatter(x, src3, dst3, zeros2, zeros1):
    mesh = plsc.VectorSubcoreMesh(core_axis_name="c", subcore_axis_name="s")

    @functools.partial(
        pl.kernel,
        mesh=mesh,
        out_type=[
            jax.ShapeDtypeStruct((NC, P, D), jnp.float32),
            jax.ShapeDtypeStruct((NW, P), jnp.float32),
        ],
        scratch_types=[
            pltpu.VMEM_SHARED((P, D), jnp.float32),
            pltpu.VMEM((NCHUNK, CH), jnp.int32),
            pltpu.VMEM((NCHUNK, CH), jnp.int32),
            pltpu.VMEM((CH, D), jnp.float32),
            pltpu.VMEM((P,), jnp.float32),
            pltpu.SemaphoreType.DMA,
        ],
    )
    def k(x_hbm, src_hbm, dst_hbm, z2_hbm, z1_hbm, psum_hbm, degs_hbm,
          acc, srcb, dstb, rowb, degb, sem):
        c = lax.axis_index("c")
        s = lax.axis_index("s")
        wid = s * NC + c
        pltpu.sync_copy(z2_hbm.at[pl.ds(s * RPT, RPT)],
                        acc.at[pl.ds(s * RPT, RPT)])
        pltpu.sync_copy(z1_hbm, degb)
        pltpu.sync_copy(src_hbm.at[wid], srcb)
        pltpu.sync_copy(dst_hbm.at[wid], dstb)
        plsc.subcore_barrier()

        ones = jnp.full((L,), 1.0, jnp.float32)

        def chunk(g, carry):
            pltpu.async_copy(x_hbm.at[srcb.at[g]], rowb, sem).wait()
            pltpu.sync_copy(rowb, acc.at[dstb.at[g]], add=True)
            for j in range(CH // L):
                idx = dstb[g, pl.ds(j * L, L)]
                plsc.addupdate_scatter(degb, [idx], ones)
            return carry

        lax.fori_loop(0, NCHUNK, chunk, 0)
        plsc.subcore_barrier()
        pltpu.sync_copy(acc.at[pl.ds(s * RPT, RPT)],
                        psum_hbm.at[c, pl.ds(s * RPT, RPT)])
        pltpu.sync_copy(degb, degs_hbm.at[wid])

    return k(x, src3, dst3, zeros2, zeros1)


BR = 400


def _combine(psum, degs):
    def body(p_ref, d_ref, o_ref):
        p = p_ref[...]
        d = jnp.sum(d_ref[...], axis=0)
        o_ref[...] = (p[0] + p[1]) / jnp.maximum(d, 1.0)[:, None]

    return pl.pallas_call(
        body,
        grid=(N_NODES // BR,),
        in_specs=[
            pl.BlockSpec((NC, BR, D), lambda i: (0, i, 0)),
            pl.BlockSpec((NW, BR), lambda i: (0, i)),
        ],
        out_specs=pl.BlockSpec((BR, D), lambda i: (i, 0)),
        out_shape=jax.ShapeDtypeStruct((N_NODES, D), jnp.float32),
    )(psum, degs)


def kernel(x, edge_index):
    ei = edge_index.astype(jnp.int32)
    pad = E_PAD - N_EDGES
    src = jnp.pad(ei[0], (0, pad)).reshape(NW, NCHUNK, CH)
    dst = jnp.pad(ei[1], (0, pad), constant_values=N_NODES).reshape(NW, NCHUNK, CH)
    zeros2 = jnp.zeros((P, D), jnp.float32)
    zeros1 = jnp.zeros((P,), jnp.float32)
    psum, degs = _sc_scatter(x, src, dst, zeros2, zeros1)
    return _combine(psum, degs)


# speedup vs baseline: 5.2544x; 5.2544x over previous
import functools

import jax
import jax.numpy as jnp
from jax import lax
from jax.experimental import pallas as pl
from jax.experimental.pallas import tpu as pltpu
from jax.experimental.pallas import tpu_sc as plsc

N_NODES = 10000
D = 128
N_EDGES = 320000
NC = 2
NS = 16
NW = NC * NS
L = 16
CH = 128
NCHUNK = (N_EDGES + NW * CH - 1) // (NW * CH)
EPT = NCHUNK * CH
E_PAD = NW * EPT
P = 10112
RPT = P // NS


def _sc_scatter(x, src3, dst3, zeros2, zeros1):
    mesh = plsc.VectorSubcoreMesh(core_axis_name="c", subcore_axis_name="s")

    @functools.partial(
        pl.kernel,
        mesh=mesh,
        out_type=[
            jax.ShapeDtypeStruct((NC, P, D), jnp.float32),
            jax.ShapeDtypeStruct((NW, P), jnp.float32),
        ],
        scratch_types=[
            pltpu.VMEM_SHARED((P, D), jnp.float32),
            pltpu.VMEM((NCHUNK, CH), jnp.int32),
            pltpu.VMEM((NCHUNK, CH), jnp.int32),
            pltpu.VMEM((CH, D), jnp.float32),
            pltpu.VMEM((P,), jnp.float32),
            pltpu.SemaphoreType.DMA,
        ],
        compiler_params=pltpu.CompilerParams(needs_layout_passes=False),
    )
    def k(x_hbm, src_hbm, dst_hbm, z2_hbm, z1_hbm, psum_hbm, degs_hbm,
          acc, srcb, dstb, rowb, degb, sem):
        c = lax.axis_index("c")
        s = lax.axis_index("s")
        wid = s * NC + c
        pltpu.sync_copy(z2_hbm.at[pl.ds(s * RPT, RPT)],
                        acc.at[pl.ds(s * RPT, RPT)])
        pltpu.sync_copy(z1_hbm, degb)
        pltpu.sync_copy(src_hbm.at[wid], srcb)
        pltpu.sync_copy(dst_hbm.at[wid], dstb)
        plsc.subcore_barrier()

        ones = jnp.full((L,), 1.0, jnp.float32)

        def chunk(g, carry):
            pltpu.async_copy(x_hbm.at[srcb.at[g]], rowb, sem).wait()
            pltpu.sync_copy(rowb, acc.at[dstb.at[g]], add=True)
            for j in range(CH // L):
                idx = dstb[g, pl.ds(j * L, L)]
                plsc.addupdate_scatter(degb, [idx], ones)
            return carry

        lax.fori_loop(0, NCHUNK, chunk, 0)
        plsc.subcore_barrier()
        pltpu.sync_copy(acc.at[pl.ds(s * RPT, RPT)],
                        psum_hbm.at[c, pl.ds(s * RPT, RPT)])
        pltpu.sync_copy(degb, degs_hbm.at[wid])

    return k(x, src3, dst3, zeros2, zeros1)


BR = 128


def _combine(psum, degs):
    def body(p_ref, d_ref, o_ref):
        p = p_ref[...]
        d = jnp.sum(d_ref[...], axis=0)
        o_ref[...] = (p[0] + p[1]) / jnp.maximum(d, 1.0)[:, None]

    return pl.pallas_call(
        body,
        grid=(P // BR,),
        in_specs=[
            pl.BlockSpec((NC, BR, D), lambda i: (0, i, 0)),
            pl.BlockSpec((NW, BR), lambda i: (0, i)),
        ],
        out_specs=pl.BlockSpec((BR, D), lambda i: (i, 0)),
        out_shape=jax.ShapeDtypeStruct((P, D), jnp.float32),
    )(psum, degs)


def kernel(x, edge_index):
    ei = edge_index.astype(jnp.int32)
    pad = E_PAD - N_EDGES
    src = jnp.pad(ei[0], (0, pad)).reshape(NW, NCHUNK, CH)
    dst = jnp.pad(ei[1], (0, pad), constant_values=N_NODES).reshape(NW, NCHUNK, CH)
    zeros2 = jnp.zeros((P, D), jnp.float32)
    zeros1 = jnp.zeros((P,), jnp.float32)
    psum, degs = _sc_scatter(x, src, dst, zeros2, zeros1)
    return _combine(psum, degs)[:N_NODES]
